# baseline (speedup 1.0000x reference)
import jax
import jax.numpy as jnp
from jax import lax
from jax.experimental import pallas as pl

_NUM_MODELS = 64
_BLK = 16384


def _mlp_body(x_ref, w0_ref, b0_ref, w1_ref, b1_ref, w2_ref, b2_ref,
              w3_ref, b3_ref, w4_ref, b4_ref, y_ref):
    bf = jnp.bfloat16
    x = x_ref[...].astype(bf)
    h = lax.dot_general(w0_ref[...], x, (((1,), (1,)), ((), ())),
                        preferred_element_type=jnp.float32) + b0_ref[...]
    h = jnp.maximum(h, 0.0).astype(bf)
    h = jnp.dot(w1_ref[...], h, preferred_element_type=jnp.float32) + b1_ref[...]
    h = jnp.maximum(h, 0.0).astype(bf)
    h = jnp.dot(w2_ref[...], h, preferred_element_type=jnp.float32) + b2_ref[...]
    h = jnp.maximum(h, 0.0).astype(bf)
    h = jnp.dot(w3_ref[...], h, preferred_element_type=jnp.float32) + b3_ref[...]
    h = jnp.maximum(h, 0.0).astype(bf)
    y_ref[...] = (jnp.dot(w4_ref[...], h, preferred_element_type=jnp.float32)
                  + b4_ref[...])


def kernel(inputs, lods, W0, b0, W1, b1, W2, b2, W3, b3, W4, b4):
    bsz, in_f = inputs.shape
    hid = W1.shape[-1]
    out_f = W4.shape[-1]
    n_lods = int(lods.shape[0])
    grid = (bsz // _BLK,)

    bf = jnp.bfloat16
    w0t = W0[0].T.astype(bf)
    w1t, w2t, w3t = (W1[0].T.astype(bf), W2[0].T.astype(bf),
                     W3[0].T.astype(bf))
    w4t = W4[0].T.astype(bf)
    b0c = b0[0][:, None]
    b1c, b2c, b3c = b1[0][:, None], b2[0][:, None], b3[0][:, None]
    b4c = b4[0][:, None]

    full = lambda shape: pl.BlockSpec(shape, lambda i: (0,) * len(shape))
    yT = pl.pallas_call(
        _mlp_body,
        grid=grid,
        in_specs=[
            pl.BlockSpec((_BLK, in_f), lambda i: (i, 0)),
            full((hid, in_f)), full((hid, 1)),
            full((hid, hid)), full((hid, 1)),
            full((hid, hid)), full((hid, 1)),
            full((hid, hid)), full((hid, 1)),
            full((out_f, hid)), full((out_f, 1)),
        ],
        out_specs=pl.BlockSpec((out_f, _BLK), lambda i: (0, i)),
        out_shape=jax.ShapeDtypeStruct((out_f, bsz), jnp.float32),
    )(inputs, w0t, b0c, w1t, b1c, w2t, b2c, w3t, b3c, w4t, b4c)

    y = yT.T
    model_outputs = jnp.broadcast_to(y[:, None, :], (bsz, n_lods, out_f))
    sel_idx = jnp.zeros((bsz,), jnp.int32)
    logit_row = jnp.concatenate(
        [jnp.zeros((1,), inputs.dtype),
         jnp.full((_NUM_MODELS - 1,), -999.9, inputs.dtype)])
    logits = jnp.broadcast_to(logit_row[None, :], (bsz, _NUM_MODELS))
    prob_row = jnp.concatenate(
        [jnp.ones((1,), inputs.dtype),
         jnp.zeros((_NUM_MODELS - 1,), inputs.dtype)])
    probs = jnp.broadcast_to(prob_row[None, :], (bsz, _NUM_MODELS))
    return (model_outputs, sel_idx, logits, probs)


# speedup vs baseline: 1.6590x; 1.6590x over previous
import jax
import jax.numpy as jnp
from jax import lax
from jax.experimental import pallas as pl

_NUM_MODELS = 64
_BLK = 32768


def _mlp_body(x_ref, w0_ref, b0_ref, w1_ref, b1_ref, w2_ref, b2_ref,
              w3_ref, b3_ref, w4_ref, b4_ref, y_ref):
    bf = jnp.bfloat16
    x = x_ref[...].astype(bf)
    h = jnp.dot(w0_ref[...], x, preferred_element_type=jnp.float32)
    h = jnp.maximum(h.astype(bf) + b0_ref[...], 0.0)
    h = jnp.dot(w1_ref[...], h, preferred_element_type=jnp.float32)
    h = jnp.maximum(h.astype(bf) + b1_ref[...], 0.0)
    h = jnp.dot(w2_ref[...], h, preferred_element_type=jnp.float32)
    h = jnp.maximum(h.astype(bf) + b2_ref[...], 0.0)
    h = jnp.dot(w3_ref[...], h, preferred_element_type=jnp.float32)
    h = jnp.maximum(h.astype(bf) + b3_ref[...], 0.0)
    y_ref[...] = (jnp.dot(w4_ref[...], h, preferred_element_type=jnp.float32)
                  + b4_ref[...])


def kernel(inputs, lods, W0, b0, W1, b1, W2, b2, W3, b3, W4, b4):
    bsz, in_f = inputs.shape
    hid = W1.shape[-1]
    out_f = W4.shape[-1]
    n_lods = int(lods.shape[0])
    grid = (bsz // _BLK,)

    bf = jnp.bfloat16
    w0t = W0[0].T.astype(bf)
    w1t, w2t, w3t = (W1[0].T.astype(bf), W2[0].T.astype(bf),
                     W3[0].T.astype(bf))
    w4t = W4[0].T.astype(bf)
    b0c = b0[0][:, None].astype(bf)
    b1c, b2c, b3c = (b1[0][:, None].astype(bf), b2[0][:, None].astype(bf),
                     b3[0][:, None].astype(bf))
    b4c = b4[0][:, None]

    full = lambda shape: pl.BlockSpec(shape, lambda i: (0,) * len(shape))
    yT = pl.pallas_call(
        _mlp_body,
        grid=grid,
        in_specs=[
            pl.BlockSpec((in_f, _BLK), lambda i: (0, i)),
            full((hid, in_f)), full((hid, 1)),
            full((hid, hid)), full((hid, 1)),
            full((hid, hid)), full((hid, 1)),
            full((hid, hid)), full((hid, 1)),
            full((out_f, hid)), full((out_f, 1)),
        ],
        out_specs=pl.BlockSpec((out_f, _BLK), lambda i: (0, i)),
        out_shape=jax.ShapeDtypeStruct((out_f, bsz), jnp.float32),
    )(inputs.T, w0t, b0c, w1t, b1c, w2t, b2c, w3t, b3c, w4t, b4c)

    y = yT.T
    model_outputs = jnp.broadcast_to(y[:, None, :], (bsz, n_lods, out_f))
    sel_idx = jnp.zeros((bsz,), jnp.int32)
    logit_row = jnp.concatenate(
        [jnp.zeros((1,), inputs.dtype),
         jnp.full((_NUM_MODELS - 1,), -999.9, inputs.dtype)])
    logits = jnp.broadcast_to(logit_row[None, :], (bsz, _NUM_MODELS))
    prob_row = jnp.concatenate(
        [jnp.ones((1,), inputs.dtype),
         jnp.zeros((_NUM_MODELS - 1,), inputs.dtype)])
    probs = jnp.broadcast_to(prob_row[None, :], (bsz, _NUM_MODELS))
    return (model_outputs, sel_idx, logits, probs)
